# baseline (device time: 300736 ns/iter reference)
import jax
import jax.numpy as jnp
from jax import lax
from jax.experimental import pallas as pl
from jax.experimental.pallas import tpu as pltpu

NZ = 4
P = 576
N_COL = 1024


def _exchange(send_buf, cnt_msg):

    def body(send_ref, cnt_ref, out_data_ref, out_cnt_ref,
             data_scratch, cnt_scratch,
             data_send_sems, data_recv_sems, cnt_send_sems, cnt_recv_sems):
        my_x = lax.axis_index("x")
        my_y = lax.axis_index("y")
        my_z = lax.axis_index("z")

        barrier_sem = pltpu.get_barrier_semaphore()
        for d in range(1, NZ):
            pl.semaphore_signal(
                barrier_sem, inc=1,
                device_id=(my_x, my_y, (my_z + d) % NZ),
                device_id_type=pl.DeviceIdType.MESH,
            )
        pl.semaphore_wait(barrier_sem, NZ - 1)

        rdmas = []
        for d in range(1, NZ):
            peer = (my_z + d) % NZ
            data_rdma = pltpu.make_async_remote_copy(
                src_ref=send_ref.at[peer],
                dst_ref=data_scratch.at[d - 1],
                send_sem=data_send_sems.at[d - 1],
                recv_sem=data_recv_sems.at[d - 1],
                device_id=(my_x, my_y, peer),
                device_id_type=pl.DeviceIdType.MESH,
            )
            data_rdma.start()
            cnt_rdma = pltpu.make_async_remote_copy(
                src_ref=cnt_ref,
                dst_ref=cnt_scratch.at[d - 1],
                send_sem=cnt_send_sems.at[d - 1],
                recv_sem=cnt_recv_sems.at[d - 1],
                device_id=(my_x, my_y, peer),
                device_id_type=pl.DeviceIdType.MESH,
            )
            cnt_rdma.start()
            rdmas.append((data_rdma, cnt_rdma))
        for data_rdma, cnt_rdma in rdmas:
            data_rdma.wait()
            cnt_rdma.wait()

        out_data_ref[...] = data_scratch[...]
        out_cnt_ref[...] = cnt_scratch[...]

    return pl.pallas_call(
        body,
        out_shape=(
            jax.ShapeDtypeStruct((NZ - 1, P, N_COL), jnp.bfloat16),
            jax.ShapeDtypeStruct((NZ - 1, 8, 128), jnp.int32),
        ),
        in_specs=[
            pl.BlockSpec(memory_space=pltpu.VMEM),
            pl.BlockSpec(memory_space=pltpu.VMEM),
        ],
        out_specs=(
            pl.BlockSpec(memory_space=pltpu.VMEM),
            pl.BlockSpec(memory_space=pltpu.VMEM),
        ),
        scratch_shapes=[
            pltpu.VMEM((NZ - 1, P, N_COL), jnp.bfloat16),
            pltpu.VMEM((NZ - 1, 8, 128), jnp.int32),
            pltpu.SemaphoreType.DMA((NZ - 1,)),
            pltpu.SemaphoreType.DMA((NZ - 1,)),
            pltpu.SemaphoreType.DMA((NZ - 1,)),
            pltpu.SemaphoreType.DMA((NZ - 1,)),
        ],
        compiler_params=pltpu.CompilerParams(collective_id=0),
    )(send_buf, cnt_msg)


def kernel(x, dest):
    n, ncol = x.shape
    my_z = lax.axis_index("z")
    dest = dest.astype(jnp.int32)

    perm = jnp.argsort(dest, stable=True).astype(jnp.int32)
    counts = jnp.sum(
        dest[None, :] == jnp.arange(NZ, dtype=jnp.int32)[:, None], axis=1
    ).astype(jnp.int32)
    offs = jnp.concatenate(
        [jnp.zeros((1,), jnp.int32), jnp.cumsum(counts)[:-1].astype(jnp.int32)]
    )

    tt = jnp.arange(NZ, dtype=jnp.int32)[:, None]
    ii = jnp.arange(P, dtype=jnp.int32)[None, :]
    valid = ii < counts[tt]
    sorted_pos = jnp.where(valid, offs[tt] + ii, 0)
    rows = jnp.where(valid, perm[sorted_pos], n)
    x_ext = jnp.concatenate([x, jnp.zeros((1, ncol), x.dtype)], axis=0)
    send_buf = x_ext[rows].astype(jnp.bfloat16)
    cnt_msg = jnp.zeros((8, 128), jnp.int32).at[0, :NZ].set(counts)

    recv_data, recv_cnt = _exchange(send_buf, cnt_msg)

    crows = jnp.concatenate([counts[None, :], recv_cnt[:, 0, :NZ]], axis=0)
    d_of_src = (my_z - jnp.arange(NZ, dtype=jnp.int32)) % NZ
    c_src = crows[d_of_src]
    lens = c_src[:, my_z]
    cum = jnp.cumsum(lens).astype(jnp.int32)
    starts = jnp.concatenate([jnp.zeros((1,), jnp.int32), cum[:-1]])

    j = jnp.arange(n, dtype=jnp.int32)
    s_j = jnp.searchsorted(cum, j, side="right").astype(jnp.int32)
    i_j = j - starts[s_j]
    d_j = (my_z - s_j) % NZ
    own = s_j == my_z
    own_rows = perm[jnp.where(own, offs[my_z] + i_j, 0)]
    rows_out = jnp.where(own, (NZ - 1) * P + own_rows, (d_j - 1) * P + i_j)
    src_concat = jnp.concatenate(
        [recv_data.reshape((NZ - 1) * P, ncol), x.astype(jnp.bfloat16)], axis=0
    )
    return src_concat[rows_out]


# device time: 83448 ns/iter; 3.6039x vs baseline; 3.6039x over previous
import jax
import jax.numpy as jnp
from jax import lax
from jax.experimental import pallas as pl
from jax.experimental.pallas import tpu as pltpu

NZ = 4
P = 576
N_COL = 1024
N_ROWS = 2048


def _a2av(x, v, cnt_msg):
    n, ncol = x.shape

    def body(x_ref, v_ref, cnt_ref, out_ref,
             send_ref, recv_ref, cnt_recv_ref,
             data_send_sems, data_recv_sems, cnt_send_sems, cnt_recv_sems):
        my_x = lax.axis_index("x")
        my_y = lax.axis_index("y")
        my_z = lax.axis_index("z")

        barrier_sem = pltpu.get_barrier_semaphore()
        for d in range(1, NZ):
            pl.semaphore_signal(
                barrier_sem, inc=1,
                device_id=(my_x, my_y, (my_z + d) % NZ),
                device_id_type=pl.DeviceIdType.MESH,
            )
        pl.semaphore_wait(barrier_sem, NZ - 1)

        xbf = x_ref[...].astype(jnp.bfloat16)
        sel = (v_ref[...] == lax.broadcasted_iota(jnp.int32, (NZ * P, n), 1))
        send = jnp.dot(sel.astype(jnp.bfloat16), xbf,
                       preferred_element_type=jnp.float32)
        send_ref[...] = send.astype(jnp.bfloat16).reshape(NZ, P, ncol)

        recv_ref[0] = send_ref[my_z]

        rdmas = []
        for d in range(1, NZ):
            peer = (my_z + d) % NZ
            data_rdma = pltpu.make_async_remote_copy(
                src_ref=send_ref.at[peer],
                dst_ref=recv_ref.at[d],
                send_sem=data_send_sems.at[d - 1],
                recv_sem=data_recv_sems.at[d - 1],
                device_id=(my_x, my_y, peer),
                device_id_type=pl.DeviceIdType.MESH,
            )
            data_rdma.start()
            cnt_rdma = pltpu.make_async_remote_copy(
                src_ref=cnt_ref,
                dst_ref=cnt_recv_ref.at[d - 1],
                send_sem=cnt_send_sems.at[d - 1],
                recv_sem=cnt_recv_sems.at[d - 1],
                device_id=(my_x, my_y, peer),
                device_id_type=pl.DeviceIdType.MESH,
            )
            cnt_rdma.start()
            rdmas.append((data_rdma, cnt_rdma))
        for data_rdma, cnt_rdma in rdmas:
            data_rdma.wait()
            cnt_rdma.wait()

        col_mask = lax.broadcasted_iota(jnp.int32, (8, 128), 1) == my_z
        row_mask = lax.broadcasted_iota(jnp.int32, (8, 128), 0) == 0
        mask = col_mask & row_mask

        def _len_of(plane):
            return jnp.sum(jnp.where(mask, plane, 0))

        l_by_d = [_len_of(cnt_ref[...])] + [
            _len_of(cnt_recv_ref[d - 1]) for d in range(1, NZ)
        ]

        len_src, slot_src = [], []
        for s in range(NZ):
            d_s = (my_z - s) % NZ
            ln = l_by_d[0]
            for d in range(1, NZ):
                ln = jnp.where(d_s == d, l_by_d[d], ln)
            len_src.append(ln)
            slot_src.append(d_s)
        starts = [jnp.int32(0)]
        for s in range(1, NZ):
            starts.append(starts[s - 1] + len_src[s - 1])

        j2 = lax.broadcasted_iota(jnp.int32, (n, 1), 0)
        s_idx = jnp.zeros((n, 1), jnp.int32)
        for s in range(1, NZ):
            s_idx = s_idx + (j2 >= starts[s]).astype(jnp.int32)
        start_j = jnp.full((n, 1), starts[0], jnp.int32)
        slot_j = jnp.full((n, 1), slot_src[0], jnp.int32)
        for s in range(1, NZ):
            sel_s = s_idx == s
            start_j = jnp.where(sel_s, starts[s], start_j)
            slot_j = jnp.where(sel_s, slot_src[s], slot_j)
        col_j = slot_j * P + (j2 - start_j)

        gsel = (col_j == lax.broadcasted_iota(jnp.int32, (n, NZ * P), 1))
        rflat = recv_ref[...].reshape(NZ * P, ncol)
        out = jnp.dot(gsel.astype(jnp.bfloat16), rflat,
                      preferred_element_type=jnp.float32)
        out_ref[...] = out.astype(jnp.bfloat16)

    return pl.pallas_call(
        body,
        out_shape=jax.ShapeDtypeStruct((n, ncol), jnp.bfloat16),
        in_specs=[
            pl.BlockSpec(memory_space=pltpu.VMEM),
            pl.BlockSpec(memory_space=pltpu.VMEM),
            pl.BlockSpec(memory_space=pltpu.VMEM),
        ],
        out_specs=pl.BlockSpec(memory_space=pltpu.VMEM),
        scratch_shapes=[
            pltpu.VMEM((NZ, P, N_COL), jnp.bfloat16),
            pltpu.VMEM((NZ, P, N_COL), jnp.bfloat16),
            pltpu.VMEM((NZ - 1, 8, 128), jnp.int32),
            pltpu.SemaphoreType.DMA((NZ - 1,)),
            pltpu.SemaphoreType.DMA((NZ - 1,)),
            pltpu.SemaphoreType.DMA((NZ - 1,)),
            pltpu.SemaphoreType.DMA((NZ - 1,)),
        ],
        compiler_params=pltpu.CompilerParams(collective_id=0),
    )(x, v, cnt_msg)


def kernel(x, dest):
    n, ncol = x.shape
    dest = dest.astype(jnp.int32)

    perm = jnp.argsort(dest, stable=True).astype(jnp.int32)
    counts = jnp.sum(
        dest[None, :] == jnp.arange(NZ, dtype=jnp.int32)[:, None], axis=1
    ).astype(jnp.int32)
    offs = jnp.concatenate(
        [jnp.zeros((1,), jnp.int32), jnp.cumsum(counts)[:-1].astype(jnp.int32)]
    )

    tt = jnp.arange(NZ, dtype=jnp.int32)[:, None]
    ii = jnp.arange(P, dtype=jnp.int32)[None, :]
    valid = ii < counts[tt]
    sorted_pos = jnp.where(valid, offs[tt] + ii, 0)
    v = jnp.where(valid, perm[sorted_pos], n).reshape(NZ * P, 1)
    cnt_msg = jnp.zeros((8, 128), jnp.int32).at[0, :NZ].set(counts)

    return _a2av(x, v, cnt_msg)


# device time: 78372 ns/iter; 3.8373x vs baseline; 1.0648x over previous
import jax
import jax.numpy as jnp
from jax import lax
from jax.experimental import pallas as pl
from jax.experimental.pallas import tpu as pltpu

NZ = 4
P = 576
N_COL = 1024
N_ROWS = 2048


def _a2av(x, dest_col):
    n, ncol = x.shape

    def body(x_ref, dest_ref, out_ref,
             send_ref, recv_ref, cnt_send_ref, cnt_recv_ref,
             data_send_sems, data_recv_sems, cnt_send_sems, cnt_recv_sems):
        my_x = lax.axis_index("x")
        my_y = lax.axis_index("y")
        my_z = lax.axis_index("z")

        barrier_sem = pltpu.get_barrier_semaphore()
        for d in range(1, NZ):
            pl.semaphore_signal(
                barrier_sem, inc=1,
                device_id=(my_x, my_y, (my_z + d) % NZ),
                device_id_type=pl.DeviceIdType.MESH,
            )
        pl.semaphore_wait(barrier_sem, NZ - 1)

        dest_c = dest_ref[...]
        mask = dest_c == lax.broadcasted_iota(jnp.int32, (n, 128), 1)
        tri = (lax.broadcasted_iota(jnp.int32, (n, n), 1)
               < lax.broadcasted_iota(jnp.int32, (n, n), 0))
        cum = jnp.dot(tri.astype(jnp.bfloat16), mask.astype(jnp.bfloat16),
                      preferred_element_type=jnp.float32)
        rank_c = jnp.sum(jnp.where(mask, cum, 0.0), axis=1, keepdims=True
                         ).astype(jnp.int32)
        counts_row = jnp.sum(mask.astype(jnp.float32), axis=0, keepdims=True
                             ).astype(jnp.int32)
        cnt_send_ref[...] = jnp.where(
            lax.broadcasted_iota(jnp.int32, (8, 128), 0) == 0, counts_row, 0)

        slot_c = dest_c * P + rank_c
        selT = (slot_c == lax.broadcasted_iota(jnp.int32, (n, NZ * P), 1))
        xbf = x_ref[...].astype(jnp.bfloat16)
        send = lax.dot_general(
            selT.astype(jnp.bfloat16), xbf,
            dimension_numbers=(((0,), (0,)), ((), ())),
            preferred_element_type=jnp.float32)
        send_ref[...] = send.astype(jnp.bfloat16).reshape(NZ, P, ncol)

        recv_ref[0] = send_ref[my_z]

        rdmas = []
        for d in range(1, NZ):
            peer = (my_z + d) % NZ
            data_rdma = pltpu.make_async_remote_copy(
                src_ref=send_ref.at[peer],
                dst_ref=recv_ref.at[d],
                send_sem=data_send_sems.at[d - 1],
                recv_sem=data_recv_sems.at[d - 1],
                device_id=(my_x, my_y, peer),
                device_id_type=pl.DeviceIdType.MESH,
            )
            data_rdma.start()
            cnt_rdma = pltpu.make_async_remote_copy(
                src_ref=cnt_send_ref,
                dst_ref=cnt_recv_ref.at[d - 1],
                send_sem=cnt_send_sems.at[d - 1],
                recv_sem=cnt_recv_sems.at[d - 1],
                device_id=(my_x, my_y, peer),
                device_id_type=pl.DeviceIdType.MESH,
            )
            cnt_rdma.start()
            rdmas.append((data_rdma, cnt_rdma))
        for data_rdma, cnt_rdma in rdmas:
            data_rdma.wait()
            cnt_rdma.wait()

        col_mask = lax.broadcasted_iota(jnp.int32, (8, 128), 1) == my_z
        row_mask = lax.broadcasted_iota(jnp.int32, (8, 128), 0) == 0
        lmask = col_mask & row_mask

        def _len_of(plane):
            return jnp.sum(jnp.where(lmask, plane, 0))

        l_by_d = [_len_of(cnt_send_ref[...])] + [
            _len_of(cnt_recv_ref[d - 1]) for d in range(1, NZ)
        ]

        len_src, slot_src = [], []
        for s in range(NZ):
            d_s = (my_z - s) % NZ
            ln = l_by_d[0]
            for d in range(1, NZ):
                ln = jnp.where(d_s == d, l_by_d[d], ln)
            len_src.append(ln)
            slot_src.append(d_s)
        starts = [jnp.int32(0)]
        for s in range(1, NZ):
            starts.append(starts[s - 1] + len_src[s - 1])

        j2 = lax.broadcasted_iota(jnp.int32, (n, 1), 0)
        s_idx = jnp.zeros((n, 1), jnp.int32)
        for s in range(1, NZ):
            s_idx = s_idx + (j2 >= starts[s]).astype(jnp.int32)
        start_j = jnp.full((n, 1), starts[0], jnp.int32)
        slot_j = jnp.full((n, 1), slot_src[0], jnp.int32)
        for s in range(1, NZ):
            sel_s = s_idx == s
            start_j = jnp.where(sel_s, starts[s], start_j)
            slot_j = jnp.where(sel_s, slot_src[s], slot_j)
        col_j = slot_j * P + (j2 - start_j)

        gsel = (col_j == lax.broadcasted_iota(jnp.int32, (n, NZ * P), 1))
        rflat = recv_ref[...].reshape(NZ * P, ncol)
        out = jnp.dot(gsel.astype(jnp.bfloat16), rflat,
                      preferred_element_type=jnp.float32)
        out_ref[...] = out.astype(jnp.bfloat16)

    return pl.pallas_call(
        body,
        out_shape=jax.ShapeDtypeStruct((n, ncol), jnp.bfloat16),
        in_specs=[
            pl.BlockSpec(memory_space=pltpu.VMEM),
            pl.BlockSpec(memory_space=pltpu.VMEM),
        ],
        out_specs=pl.BlockSpec(memory_space=pltpu.VMEM),
        scratch_shapes=[
            pltpu.VMEM((NZ, P, N_COL), jnp.bfloat16),
            pltpu.VMEM((NZ, P, N_COL), jnp.bfloat16),
            pltpu.VMEM((8, 128), jnp.int32),
            pltpu.VMEM((NZ - 1, 8, 128), jnp.int32),
            pltpu.SemaphoreType.DMA((NZ - 1,)),
            pltpu.SemaphoreType.DMA((NZ - 1,)),
            pltpu.SemaphoreType.DMA((NZ - 1,)),
            pltpu.SemaphoreType.DMA((NZ - 1,)),
        ],
        compiler_params=pltpu.CompilerParams(collective_id=0),
    )(x, dest_col)


def kernel(x, dest):
    n, _ = x.shape
    return _a2av(x, dest.astype(jnp.int32).reshape(n, 1))
